# Initial kernel scaffold; baseline (speedup 1.0000x reference)
import jax
import jax.numpy as jnp
from jax.experimental import pallas as pl


def kernel(points_lst):
    raise NotImplementedError("write your pallas kernel here")


# speedup vs baseline: 1.4054x; 1.4054x over previous
import functools

import numpy as np

import jax
import jax.numpy as jnp
from jax import lax
from jax.experimental import pallas as pl
from jax.experimental.pallas import tpu as pltpu
from jax.experimental.pallas import tpu_sc as plsc

F32 = jnp.float32
I32 = jnp.int32

GX, GY = 432, 496
NBINS = GX * GY
MAXV = 20000
MAXP = 30
NRAW = 150000
NP = 150016
CHUNK = NP // 32
NVEC = CHUNK // 16
NIDX = 37
STRIPE = 6720
HTOT = 32 * STRIPE
NSVEC = STRIPE // 16
FOUT = MAXV * MAXP * 4
FPAD = 64
COUT = MAXV * 4
CPAD = 64
STG = 2048

X0, X1 = np.float32(0.0), np.float32(69.12)
Y0, Y1 = np.float32(-39.68), np.float32(39.68)
Z0, Z1 = np.float32(-3.0), np.float32(1.0)
VS = np.float32(0.16)

_MESH = None


def _mesh():
    global _MESH
    if _MESH is None:
        _MESH = plsc.VectorSubcoreMesh(
            core_axis_name="c", subcore_axis_name="s", num_cores=2,
            num_subcores=16)
    return _MESH


def _wid():
    return lax.axis_index("s") * 2 + lax.axis_index("c")


def _k1_body(xs, ys, zs, vid_hbm, cnta_hbm, cntb_hbm,
             xv, yv, zv, vidv, idx2, ones, zbuf, hist):
    c = lax.axis_index("c")
    s = lax.axis_index("s")
    w = s * 2 + c
    iota = lax.iota(I32, 16)
    base = w * CHUNK

    nz = zbuf.shape[0]
    for q in range(16):
        zbuf[pl.ds(q * 16, 16)] = iota * 0

    def zloop(q, _):
        pltpu.sync_copy(zbuf, hist.at[pl.ds(s * (HTOT // 16) + q * nz, nz)])
        return 0
    lax.fori_loop(0, (HTOT // 16) // nz, zloop, 0)

    for q in range(8):
        ones[pl.ds(q * 16, 16)] = iota * 0 + 1

    pltpu.sync_copy(xs.at[pl.ds(base, CHUNK)], xv)
    pltpu.sync_copy(ys.at[pl.ds(base, CHUNK)], yv)
    pltpu.sync_copy(zs.at[pl.ds(base, CHUNK)], zv)

    for q in range(8):
        p = NVEC * 16 + q * 16 + iota
        pm = p < NIDX * 128
        tgt = NBINS + 16 + (p & 63)
        plsc.store_scatter(idx2, [p >> 7, p & 127], tgt, mask=pm)

    def body(j, _):
        x = xv[pl.ds(j * 16, 16)]
        y = yv[pl.ds(j * 16, 16)]
        z = zv[pl.ds(j * 16, 16)]
        inr = ((x >= X0) & (x < X1) & (y >= Y0) & (y < Y1)
               & (z >= Z0) & (z < Z1))
        cx = jnp.clip(((x - X0) / VS).astype(I32), 0, GX - 1)
        cy = jnp.clip(((y - Y0) / VS).astype(I32), 0, GY - 1)
        v = jnp.where(inr, cy * GX + cx, NBINS)
        vidv[pl.ds(j * 16, 16)] = v
        p = j * 16 + iota
        plsc.store_scatter(idx2, [p >> 7, p & 127], v)
        return 0
    lax.fori_loop(0, NVEC, body, 0)

    plsc.subcore_barrier()

    def hloop(g, _):
        pltpu.sync_copy(ones, hist.at[idx2.at[g]], add=True)
        return 0
    lax.fori_loop(0, NIDX, hloop, 0)

    plsc.subcore_barrier()

    pltpu.sync_copy(vidv, vid_hbm.at[pl.ds(base, CHUNK)])

    @pl.when(c == 0)
    def _():
        pltpu.sync_copy(hist.at[pl.ds(s * (HTOT // 16), HTOT // 16)],
                        cnta_hbm.at[pl.ds(s * (HTOT // 16), HTOT // 16)])

    @pl.when(c == 1)
    def _():
        pltpu.sync_copy(hist.at[pl.ds(s * (HTOT // 16), HTOT // 16)],
                        cntb_hbm.at[pl.ds(s * (HTOT // 16), HTOT // 16)])


def _k2_body(cnta_hbm, cntb_hbm, rank_hbm, tot_hbm, c0, c1, rkv, tv):
    w = _wid()
    iota = lax.iota(I32, 16)
    lo = w * STRIPE
    pltpu.sync_copy(cnta_hbm.at[pl.ds(lo, STRIPE)], c0)
    pltpu.sync_copy(cntb_hbm.at[pl.ds(lo, STRIPE)], c1)

    def body(j, carry):
        cc = c0[pl.ds(j * 16, 16)] + c1[pl.ds(j * 16, 16)]
        binid = lo + j * 16 + iota
        occ = jnp.where((cc > 0) & (binid < NBINS), 1, 0).astype(I32)
        incl = plsc.cumsum(occ)
        rkv[pl.ds(j * 16, 16)] = carry + incl - occ
        return carry + jnp.sum(occ)
    tot = lax.fori_loop(0, NSVEC, body, jnp.int32(0))

    pltpu.sync_copy(rkv, rank_hbm.at[pl.ds(lo, STRIPE)])
    tv[...] = iota * 0 + tot
    pltpu.sync_copy(tv, tot_hbm.at[w])


def _k3_body(vid_hbm, xs, ys, zs, ws, rank_hbm, tot_hbm,
             fscr, cscr, dbg_hbm,
             rkv, cntv, vch, cvid, cidx, fbx, fby, fbz, fbw,
             sidx2, sdatf, sdati, totv, t16, sem):
    w = _wid()
    iota = lax.iota(I32, 16)
    lo = w * STRIPE
    hi = jnp.minimum(lo + STRIPE, NBINS)

    pltpu.sync_copy(tot_hbm, totv)
    t0 = plsc.load_gather(totv, [iota, iota * 0])
    t1 = plsc.load_gather(totv, [iota + 16, iota * 0])
    base = (jnp.sum(jnp.where(iota < w, t0, 0))
            + jnp.sum(jnp.where(iota + 16 < w, t1, 0)))

    t16[...] = iota * 0 + base
    pltpu.sync_copy(t16, dbg_hbm.at[w])

    @pl.when(base < MAXV)
    def _():
        pltpu.sync_copy(rank_hbm.at[pl.ds(lo, STRIPE)], rkv)

        def zc(j, _):
            cntv[pl.ds(j * 16, 16)] = iota * 0
            return 0
        lax.fori_loop(0, NSVEC, zc, 0)

        def flush(sn, dat, dst, padbase):
            glast = sn >> 7
            gmax = (sn + 127) >> 7

            @pl.when(glast < 16)
            def _():
                for q in range(8):
                    p = glast * 128 + q * 16 + iota
                    plsc.store_scatter(
                        sidx2, [iota * 0 + glast, q * 16 + iota],
                        padbase + (p & 63), mask=p >= sn)

            def floop(g, _):
                pltpu.async_copy(dat.at[pl.ds(g * 128, 128)],
                                 dst.at[sidx2.at[g]], sem).wait()
                return 0
            lax.fori_loop(0, gmax, floop, 0)

        def chunk(k, _):
            cb = k * CHUNK
            pltpu.sync_copy(vid_hbm.at[pl.ds(cb, CHUNK)], vch)

            def fbody(j, na):
                v = vch[pl.ds(j * 16, 16)]
                m = (v >= lo) & (v < hi)
                plsc.store_compressed(cvid.at[pl.ds(na, 16)], v, mask=m)
                plsc.store_compressed(cidx.at[pl.ds(na, 16)],
                                      cb + j * 16 + iota, mask=m)
                return na + jnp.sum(jnp.where(m, 1, 0).astype(I32))
            na = lax.fori_loop(0, NVEC, fbody, jnp.int32(0))

            @pl.when(na > 0)
            def _():
                for q in range(8):
                    p = na + q * 16 + iota
                    pm = p < CHUNK + 128
                    plsc.store_scatter(cidx, [jnp.where(pm, p, 0)],
                                       iota * 0, mask=pm)

                ng = (na + 127) >> 7

                def gloop(g, _):
                    sl = pl.ds(g * 128, 128)
                    pltpu.async_copy(xs.at[cidx.at[sl]], fbx.at[sl],
                                     sem).wait()
                    pltpu.async_copy(ys.at[cidx.at[sl]], fby.at[sl],
                                     sem).wait()
                    pltpu.async_copy(zs.at[cidx.at[sl]], fbz.at[sl],
                                     sem).wait()
                    pltpu.async_copy(ws.at[cidx.at[sl]], fbw.at[sl],
                                     sem).wait()
                    return 0
                lax.fori_loop(0, ng, gloop, 0)

                nh = (na + 15) >> 4

                def hbody(j, sn):
                    p16 = j * 16 + iota
                    mval = p16 < na
                    v = cvid[pl.ds(j * 16, 16)]
                    vloc = jnp.where(mval, v - lo, 0)
                    rloc = plsc.load_gather(rkv, [vloc], mask=mval)
                    rg = rloc + base
                    keep = mval & (rg < MAXV)
                    ccur = plsc.load_gather(cntv, [vloc], mask=keep)
                    occ1, lastm = plsc.scan_count(v, mask=keep)
                    ccur = jnp.where(keep, ccur, 0)
                    occ1 = jnp.where(keep, occ1, 1)
                    pos = ccur + occ1 - 1
                    plsc.store_scatter(cntv, [vloc], ccur + occ1,
                                       mask=keep & lastm)
                    ok = keep & (pos < MAXP)
                    row4 = (rg * MAXP + pos) * 4
                    cpos = plsc.cumsum(jnp.where(ok, 1, 0).astype(I32),
                                       mask=ok)
                    slot = sn + (cpos - 1) * 4
                    x = fbx[pl.ds(j * 16, 16)]
                    y = fby[pl.ds(j * 16, 16)]
                    z = fbz[pl.ds(j * 16, 16)]
                    u = fbw[pl.ds(j * 16, 16)]
                    fx = (x - X0) / (X1 - X0)
                    fy = (y - Y0) / (Y1 - Y0)
                    fz = (z - Z0) / (Z1 - Z0)
                    for cc, val in ((0, fx), (1, fy), (2, fz), (3, u)):
                        sl = slot + cc
                        plsc.store_scatter(sidx2, [sl >> 7, sl & 127],
                                           row4 + cc, mask=ok)
                        plsc.store_scatter(sdatf, [sl], val, mask=ok)
                    sn = sn + 4 * jnp.sum(jnp.where(ok, 1, 0).astype(I32))

                    def do_flush(s_):
                        flush(s_, sdatf, fscr, FOUT)
                        return jnp.int32(0)
                    sn = lax.cond(sn > STG - 64, do_flush,
                                  lambda s_: s_, sn)
                    return sn
                sn = lax.fori_loop(0, nh, hbody, jnp.int32(0))
                lax.cond(sn > 0,
                         lambda s_: (flush(s_, sdatf, fscr, FOUT),
                                     jnp.int32(0))[1],
                         lambda s_: s_, sn)
            return 0
        lax.fori_loop(0, 32, chunk, 0)

        def cbody(j, sn):
            cnt16 = cntv[pl.ds(j * 16, 16)]
            m = cnt16 > 0
            v16 = lo + j * 16 + iota
            rg = rkv[pl.ds(j * 16, 16)] + base
            cy = v16 // GX
            cx = v16 - cy * GX
            cpos = plsc.cumsum(jnp.where(m, 1, 0).astype(I32), mask=m)
            slot = sn + (cpos - 1) * 2
            for cc, val in ((0, cy), (1, cx)):
                sl = slot + cc
                plsc.store_scatter(sidx2, [sl >> 7, sl & 127],
                                   rg * 4 + 2 + cc, mask=m)
                plsc.store_scatter(sdati, [sl], val, mask=m)
            sn = sn + 2 * jnp.sum(jnp.where(m, 1, 0).astype(I32))

            def do_flush(s_):
                flush(s_, sdati, cscr, COUT)
                return jnp.int32(0)
            sn = lax.cond(sn > STG - 32, do_flush, lambda s_: s_, sn)
            return sn
        sn = lax.fori_loop(0, NSVEC, cbody, jnp.int32(0))
        lax.cond(sn > 0,
                 lambda s_: (flush(s_, sdati, cscr, COUT),
                             jnp.int32(0))[1],
                 lambda s_: s_, sn)


def _build_calls():
    mesh = _mesh()
    k1 = pl.kernel(
        functools.partial(_k1_body),
        out_type=(
            jax.ShapeDtypeStruct((NP,), I32),
            jax.ShapeDtypeStruct((HTOT,), I32),
            jax.ShapeDtypeStruct((HTOT,), I32),
        ),
        mesh=mesh,
        scratch_types=[
            pltpu.VMEM((CHUNK,), F32),
            pltpu.VMEM((CHUNK,), F32),
            pltpu.VMEM((CHUNK,), F32),
            pltpu.VMEM((CHUNK,), I32),
            pltpu.VMEM((NIDX, 128), I32),
            pltpu.VMEM((128,), I32),
            pltpu.VMEM((1344,), I32),
            pltpu.VMEM_SHARED((HTOT,), I32),
        ],
        compiler_params=pltpu.CompilerParams(needs_layout_passes=False),
    )
    k2 = pl.kernel(
        _k2_body,
        out_type=(
            jax.ShapeDtypeStruct((HTOT,), I32),
            jax.ShapeDtypeStruct((32, 16), I32),
        ),
        mesh=mesh,
        scratch_types=[
            pltpu.VMEM((STRIPE,), I32),
            pltpu.VMEM((STRIPE,), I32),
            pltpu.VMEM((STRIPE,), I32),
            pltpu.VMEM((16,), I32),
        ],
        compiler_params=pltpu.CompilerParams(needs_layout_passes=False),
    )
    k3 = pl.kernel(
        _k3_body,
        out_type=jax.ShapeDtypeStruct((32, 16), I32),
        mesh=mesh,
        scratch_types=[
            pltpu.VMEM((STRIPE,), I32),
            pltpu.VMEM((STRIPE,), I32),
            pltpu.VMEM((CHUNK,), I32),
            pltpu.VMEM((CHUNK + 128,), I32),
            pltpu.VMEM((CHUNK + 128,), I32),
            pltpu.VMEM((CHUNK + 128,), F32),
            pltpu.VMEM((CHUNK + 128,), F32),
            pltpu.VMEM((CHUNK + 128,), F32),
            pltpu.VMEM((CHUNK + 128,), F32),
            pltpu.VMEM((16, 128), I32),
            pltpu.VMEM((STG,), F32),
            pltpu.VMEM((STG,), I32),
            pltpu.VMEM((32, 16), I32),
            pltpu.VMEM((16,), I32),
            pltpu.SemaphoreType.DMA,
        ],
        compiler_params=pltpu.CompilerParams(needs_layout_passes=False),
    )
    return k1, k2, k3


def kernel(points_lst):
    pts = points_lst
    pad_x = jnp.full((NP - NRAW,), -100.0, F32)
    pad_0 = jnp.zeros((NP - NRAW,), F32)
    xs = jnp.concatenate([pts[:, 0], pad_x])
    ys = jnp.concatenate([pts[:, 1], pad_0])
    zs = jnp.concatenate([pts[:, 2], pad_0])
    ws = jnp.concatenate([pts[:, 3], pad_0])

    k1, k2, k3 = _build_calls()
    vid, cnta, cntb = k1(xs, ys, zs)
    rank, tot = k2(cnta, cntb)

    fref = jax.new_ref(jnp.zeros((FOUT + FPAD,), F32))
    cref = jax.new_ref(jnp.zeros((COUT + CPAD,), I32))
    k3(vid, xs, ys, zs, ws, rank, tot, fref, cref)

    features = fref[...][:FOUT].reshape(MAXV, MAXP, 4)
    coors = cref[...][:COUT].reshape(MAXV, 4)
    return features, coors


# speedup vs baseline: 3.5213x; 2.5055x over previous
import jax
import jax.numpy as jnp
import numpy as np
from jax import lax
from jax.experimental import pallas as pl
from jax.experimental.pallas import tpu as pltpu
from jax.experimental.pallas import tpu_sc as plsc

F32 = jnp.float32
I32 = jnp.int32

GX, GY = 432, 496
NBINS = GX * GY
MAXV = 20000
MAXP = 30
NRAW = 150000
NP = 150016
CHUNK = NP // 32
NVEC = CHUNK // 16
NIDX = 37
STRIPE = 6720
HTOT = 32 * STRIPE
NSVEC = STRIPE // 16
FOUT = MAXV * MAXP * 4
COUT = MAXV * 4
STG = 2048

X0, X1 = np.float32(0.0), np.float32(69.12)
Y0, Y1 = np.float32(-39.68), np.float32(39.68)
Z0, Z1 = np.float32(-3.0), np.float32(1.0)
VS = np.float32(0.16)

_MESH = None


def _mesh():
    global _MESH
    if _MESH is None:
        _MESH = plsc.VectorSubcoreMesh(
            core_axis_name="c", subcore_axis_name="s", num_cores=2,
            num_subcores=16)
    return _MESH


def _k1_body(pts_hbm, vid_hbm, xs_o, ys_o, zs_o, ws_o, cnta_hbm, cntb_hbm,
             ptv, xv, yv, zv, wv, vidv, idx2, ones, zbuf, hist):
    c = lax.axis_index("c")
    s = lax.axis_index("s")
    w = s * 2 + c
    iota = lax.iota(I32, 16)
    base = w * CHUNK

    nz = zbuf.shape[0]
    for q in range(16):
        zbuf[pl.ds(q * 16, 16)] = iota * 0

    def zloop(q, _):
        pltpu.sync_copy(zbuf, hist.at[pl.ds(s * (HTOT // 16) + q * nz, nz)])
        return 0
    lax.fori_loop(0, (HTOT // 16) // nz, zloop, 0)

    for q in range(8):
        ones[pl.ds(q * 16, 16)] = iota * 0 + 1

    @pl.when(w < 31)
    def _():
        pltpu.sync_copy(pts_hbm.at[pl.ds(base * 4, CHUNK * 4)], ptv)

    @pl.when(w == 31)
    def _():
        tail = (NRAW - 31 * CHUNK) * 4
        pltpu.sync_copy(pts_hbm.at[pl.ds(31 * CHUNK * 4, tail)],
                        ptv.at[pl.ds(0, tail)])

    for q in range(8):
        p = NVEC * 16 + q * 16 + iota
        pm = p < NIDX * 128
        tgt = NBINS + 16 + (p & 63)
        plsc.store_scatter(idx2, [p >> 7, p & 127], tgt, mask=pm)

    def body(j, _):
        i4 = (j * 16 + iota) * 4
        x = plsc.load_gather(ptv, [i4])
        y = plsc.load_gather(ptv, [i4 + 1])
        z = plsc.load_gather(ptv, [i4 + 2])
        u = plsc.load_gather(ptv, [i4 + 3])
        xv[pl.ds(j * 16, 16)] = x
        yv[pl.ds(j * 16, 16)] = y
        zv[pl.ds(j * 16, 16)] = z
        wv[pl.ds(j * 16, 16)] = u
        inr = ((x >= X0) & (x < X1) & (y >= Y0) & (y < Y1)
               & (z >= Z0) & (z < Z1) & (base + j * 16 + iota < NRAW))
        cx = jnp.clip(((x - X0) / VS).astype(I32), 0, GX - 1)
        cy = jnp.clip(((y - Y0) / VS).astype(I32), 0, GY - 1)
        v = jnp.where(inr, cy * GX + cx, NBINS)
        vidv[pl.ds(j * 16, 16)] = v
        p = j * 16 + iota
        plsc.store_scatter(idx2, [p >> 7, p & 127], v)
        return 0
    lax.fori_loop(0, NVEC, body, 0)

    plsc.subcore_barrier()

    def hloop(g, _):
        pltpu.sync_copy(ones, hist.at[idx2.at[g]], add=True)
        return 0
    lax.fori_loop(0, NIDX, hloop, 0)

    plsc.subcore_barrier()

    pltpu.sync_copy(vidv, vid_hbm.at[pl.ds(base, CHUNK)])
    pltpu.sync_copy(xv, xs_o.at[pl.ds(base, CHUNK)])
    pltpu.sync_copy(yv, ys_o.at[pl.ds(base, CHUNK)])
    pltpu.sync_copy(zv, zs_o.at[pl.ds(base, CHUNK)])
    pltpu.sync_copy(wv, ws_o.at[pl.ds(base, CHUNK)])

    @pl.when(c == 0)
    def _():
        pltpu.sync_copy(hist.at[pl.ds(s * (HTOT // 16), HTOT // 16)],
                        cnta_hbm.at[pl.ds(s * (HTOT // 16), HTOT // 16)])

    @pl.when(c == 1)
    def _():
        pltpu.sync_copy(hist.at[pl.ds(s * (HTOT // 16), HTOT // 16)],
                        cntb_hbm.at[pl.ds(s * (HTOT // 16), HTOT // 16)])


def _k2_body(cnta_hbm, cntb_hbm, rank_hbm, tot_hbm, c0, c1, rkv, tv):
    c = lax.axis_index("c")
    s = lax.axis_index("s")
    w = s * 2 + c
    iota = lax.iota(I32, 16)
    lo = w * STRIPE
    pltpu.sync_copy(cnta_hbm.at[pl.ds(lo, STRIPE)], c0)
    pltpu.sync_copy(cntb_hbm.at[pl.ds(lo, STRIPE)], c1)

    def body(j, carry):
        cc = c0[pl.ds(j * 16, 16)] + c1[pl.ds(j * 16, 16)]
        binid = lo + j * 16 + iota
        occ = jnp.where((cc > 0) & (binid < NBINS), 1, 0).astype(I32)
        incl = plsc.cumsum(occ)
        rkv[pl.ds(j * 16, 16)] = carry + incl - occ
        return carry + jnp.sum(occ)
    tot = lax.fori_loop(0, NSVEC, body, jnp.int32(0))

    pltpu.sync_copy(rkv, rank_hbm.at[pl.ds(lo, STRIPE)])
    tv[...] = iota * 0 + tot
    pltpu.sync_copy(tv, tot_hbm.at[w])


def _k3_body(vid_hbm, xs, ys, zs, ws, rank_hbm, tot_hbm,
             fscr, cscr, dbg_hbm,
             rkv, cntv, vchA, vchB, cvid, cidx, fbx, fby, fbz, fbw,
             sidx2, sdatf, sdati, totv, t16, semA, semB, semG, semF):
    c = lax.axis_index("c")
    s = lax.axis_index("s")
    w = s * 2 + c
    iota = lax.iota(I32, 16)
    lo = w * STRIPE
    hi = jnp.minimum(lo + STRIPE, NBINS)

    pltpu.sync_copy(tot_hbm, totv)
    t0 = plsc.load_gather(totv, [iota, iota * 0])
    t1 = plsc.load_gather(totv, [iota + 16, iota * 0])
    base = (jnp.sum(jnp.where(iota < w, t0, 0))
            + jnp.sum(jnp.where(iota + 16 < w, t1, 0)))

    t16[...] = iota * 0 + base
    pltpu.sync_copy(t16, dbg_hbm.at[w])

    @pl.when(base < MAXV)
    def _():
        pltpu.sync_copy(rank_hbm.at[pl.ds(lo, STRIPE)], rkv)

        def zc(j, _):
            cntv[pl.ds(j * 16, 16)] = iota * 0
            return 0
        lax.fori_loop(0, NSVEC, zc, 0)

        def flush(sn, dat, dst):
            glast = sn >> 7
            gmax = (sn + 127) >> 7
            lastq = (sn - 1) >> 7
            lastr = (sn - 1) & 127
            ilast = plsc.load_gather(sidx2, [iota * 0 + lastq,
                                             iota * 0 + lastr])
            vlast = plsc.load_gather(dat, [iota * 0 + (sn - 1)])

            @pl.when(glast < 16)
            def _():
                for q in range(8):
                    p = glast * 128 + q * 16 + iota
                    pm = p >= sn
                    plsc.store_scatter(sidx2, [iota * 0 + glast,
                                               q * 16 + iota], ilast,
                                       mask=pm)
                    plsc.store_scatter(dat, [p], vlast, mask=pm)

            def fire(g, _):
                pltpu.async_copy(dat.at[pl.ds(g * 128, 128)],
                                 dst.at[sidx2.at[g]], semF)
                return 0
            lax.fori_loop(0, gmax, fire, 0)

            def drain(g, _):
                pltpu.make_async_copy(dat.at[pl.ds(g * 128, 128)],
                                      dst.at[sidx2.at[g]], semF).wait()
                return 0
            lax.fori_loop(0, gmax, drain, 0)

        def process(kchunk, vch, semV):
            cb = kchunk * CHUNK
            pltpu.make_async_copy(vid_hbm.at[pl.ds(cb, CHUNK)], vch,
                                  semV).wait()

            def fbody(j, na):
                v = vch[pl.ds(j * 16, 16)]
                m = (v >= lo) & (v < hi)
                plsc.store_compressed(cvid.at[pl.ds(na, 16)], v, mask=m)
                plsc.store_compressed(cidx.at[pl.ds(na, 16)],
                                      cb + j * 16 + iota, mask=m)
                return na + jnp.sum(jnp.where(m, 1, 0).astype(I32))
            na = lax.fori_loop(0, NVEC, fbody, jnp.int32(0))

            @pl.when(na > 0)
            def _():
                for q in range(8):
                    p = na + q * 16 + iota
                    pm = p < CHUNK + 128
                    plsc.store_scatter(cidx, [jnp.where(pm, p, 0)],
                                       iota * 0, mask=pm)

                ng = (na + 127) >> 7

                def gfire(g, _):
                    sl = pl.ds(g * 128, 128)
                    pltpu.async_copy(xs.at[cidx.at[sl]], fbx.at[sl], semG)
                    pltpu.async_copy(ys.at[cidx.at[sl]], fby.at[sl], semG)
                    pltpu.async_copy(zs.at[cidx.at[sl]], fbz.at[sl], semG)
                    pltpu.async_copy(ws.at[cidx.at[sl]], fbw.at[sl], semG)
                    return 0
                lax.fori_loop(0, ng, gfire, 0)

                def gdrain(g, _):
                    sl = pl.ds(g * 128, 128)
                    pltpu.make_async_copy(xs.at[cidx.at[sl]], fbx.at[sl],
                                          semG).wait()
                    pltpu.make_async_copy(ys.at[cidx.at[sl]], fby.at[sl],
                                          semG).wait()
                    pltpu.make_async_copy(zs.at[cidx.at[sl]], fbz.at[sl],
                                          semG).wait()
                    pltpu.make_async_copy(ws.at[cidx.at[sl]], fbw.at[sl],
                                          semG).wait()
                    return 0
                lax.fori_loop(0, ng, gdrain, 0)

                nh = (na + 15) >> 4

                def hbody(j, sn):
                    p16 = j * 16 + iota
                    mval = p16 < na
                    v = cvid[pl.ds(j * 16, 16)]
                    vloc = jnp.where(mval, v - lo, 0)
                    rloc = plsc.load_gather(rkv, [vloc], mask=mval)
                    rg = rloc + base
                    keep = mval & (rg < MAXV)
                    ccur = plsc.load_gather(cntv, [vloc], mask=keep)
                    occ1, lastm = plsc.scan_count(v, mask=keep)
                    ccur = jnp.where(keep, ccur, 0)
                    occ1 = jnp.where(keep, occ1, 1)
                    pos = ccur + occ1 - 1
                    plsc.store_scatter(cntv, [vloc], ccur + occ1,
                                       mask=keep & lastm)
                    ok = keep & (pos < MAXP)
                    row4 = (rg * MAXP + pos) * 4
                    cpos = plsc.cumsum(jnp.where(ok, 1, 0).astype(I32),
                                       mask=ok)
                    slot = sn + (cpos - 1) * 4
                    x = fbx[pl.ds(j * 16, 16)]
                    y = fby[pl.ds(j * 16, 16)]
                    z = fbz[pl.ds(j * 16, 16)]
                    u = fbw[pl.ds(j * 16, 16)]
                    fx = (x - X0) / (X1 - X0)
                    fy = (y - Y0) / (Y1 - Y0)
                    fz = (z - Z0) / (Z1 - Z0)
                    for cc, val in ((0, fx), (1, fy), (2, fz), (3, u)):
                        sl = slot + cc
                        plsc.store_scatter(sidx2, [sl >> 7, sl & 127],
                                           row4 + cc, mask=ok)
                        plsc.store_scatter(sdatf, [sl], val, mask=ok)
                    sn = sn + 4 * jnp.sum(jnp.where(ok, 1, 0).astype(I32))

                    def do_flush(s_):
                        flush(s_, sdatf, fscr)
                        return jnp.int32(0)
                    sn = lax.cond(sn > STG - 64, do_flush,
                                  lambda s_: s_, sn)
                    return sn
                sn = lax.fori_loop(0, nh, hbody, jnp.int32(0))
                lax.cond(sn > 0,
                         lambda s_: (flush(s_, sdatf, fscr),
                                     jnp.int32(0))[1],
                         lambda s_: s_, sn)

        pltpu.async_copy(vid_hbm.at[pl.ds(0, CHUNK)], vchA, semA)
        pltpu.async_copy(vid_hbm.at[pl.ds(CHUNK, CHUNK)], vchB, semB)

        def chunk2(i, _):
            process(2 * i, vchA, semA)

            @pl.when(2 * i + 2 < 32)
            def _():
                pltpu.async_copy(vid_hbm.at[pl.ds((2 * i + 2) * CHUNK,
                                                  CHUNK)], vchA, semA)
            process(2 * i + 1, vchB, semB)

            @pl.when(2 * i + 3 < 32)
            def _():
                pltpu.async_copy(vid_hbm.at[pl.ds((2 * i + 3) * CHUNK,
                                                  CHUNK)], vchB, semB)
            return 0
        lax.fori_loop(0, 16, chunk2, 0)

        def cbody(j, sn):
            cnt16 = cntv[pl.ds(j * 16, 16)]
            m = cnt16 > 0
            v16 = lo + j * 16 + iota
            rg = rkv[pl.ds(j * 16, 16)] + base
            cy = v16 // GX
            cx = v16 - cy * GX
            cpos = plsc.cumsum(jnp.where(m, 1, 0).astype(I32), mask=m)
            slot = sn + (cpos - 1) * 2
            for cc, val in ((0, cy), (1, cx)):
                sl = slot + cc
                plsc.store_scatter(sidx2, [sl >> 7, sl & 127],
                                   rg * 4 + 2 + cc, mask=m)
                plsc.store_scatter(sdati, [sl], val, mask=m)
            sn = sn + 2 * jnp.sum(jnp.where(m, 1, 0).astype(I32))

            def do_flush(s_):
                flush(s_, sdati, cscr)
                return jnp.int32(0)
            sn = lax.cond(sn > STG - 32, do_flush, lambda s_: s_, sn)
            return sn
        sn = lax.fori_loop(0, NSVEC, cbody, jnp.int32(0))
        lax.cond(sn > 0,
                 lambda s_: (flush(s_, sdati, cscr), jnp.int32(0))[1],
                 lambda s_: s_, sn)


def _build_calls():
    mesh = _mesh()
    k1 = pl.kernel(
        _k1_body,
        out_type=(
            jax.ShapeDtypeStruct((NP,), I32),
            jax.ShapeDtypeStruct((NP,), F32),
            jax.ShapeDtypeStruct((NP,), F32),
            jax.ShapeDtypeStruct((NP,), F32),
            jax.ShapeDtypeStruct((NP,), F32),
            jax.ShapeDtypeStruct((HTOT,), I32),
            jax.ShapeDtypeStruct((HTOT,), I32),
        ),
        mesh=mesh,
        scratch_types=[
            pltpu.VMEM((CHUNK * 4,), F32),
            pltpu.VMEM((CHUNK,), F32),
            pltpu.VMEM((CHUNK,), F32),
            pltpu.VMEM((CHUNK,), F32),
            pltpu.VMEM((CHUNK,), F32),
            pltpu.VMEM((CHUNK,), I32),
            pltpu.VMEM((NIDX, 128), I32),
            pltpu.VMEM((128,), I32),
            pltpu.VMEM((1344,), I32),
            pltpu.VMEM_SHARED((HTOT,), I32),
        ],
        compiler_params=pltpu.CompilerParams(needs_layout_passes=False),
    )
    k2 = pl.kernel(
        _k2_body,
        out_type=(
            jax.ShapeDtypeStruct((HTOT,), I32),
            jax.ShapeDtypeStruct((32, 16), I32),
        ),
        mesh=mesh,
        scratch_types=[
            pltpu.VMEM((STRIPE,), I32),
            pltpu.VMEM((STRIPE,), I32),
            pltpu.VMEM((STRIPE,), I32),
            pltpu.VMEM((16,), I32),
        ],
        compiler_params=pltpu.CompilerParams(needs_layout_passes=False),
    )
    k3 = pl.kernel(
        _k3_body,
        out_type=jax.ShapeDtypeStruct((32, 16), I32),
        mesh=mesh,
        scratch_types=[
            pltpu.VMEM((STRIPE,), I32),
            pltpu.VMEM((STRIPE,), I32),
            pltpu.VMEM((CHUNK,), I32),
            pltpu.VMEM((CHUNK,), I32),
            pltpu.VMEM((CHUNK + 128,), I32),
            pltpu.VMEM((CHUNK + 128,), I32),
            pltpu.VMEM((CHUNK + 128,), F32),
            pltpu.VMEM((CHUNK + 128,), F32),
            pltpu.VMEM((CHUNK + 128,), F32),
            pltpu.VMEM((CHUNK + 128,), F32),
            pltpu.VMEM((16, 128), I32),
            pltpu.VMEM((STG,), F32),
            pltpu.VMEM((STG,), I32),
            pltpu.VMEM((32, 16), I32),
            pltpu.VMEM((16,), I32),
            pltpu.SemaphoreType.DMA,
            pltpu.SemaphoreType.DMA,
            pltpu.SemaphoreType.DMA,
            pltpu.SemaphoreType.DMA,
        ],
        compiler_params=pltpu.CompilerParams(needs_layout_passes=False),
    )
    return k1, k2, k3


def kernel(points_lst):
    flat = points_lst.reshape(-1)
    k1, k2, k3 = _build_calls()
    vid, xs, ys, zs, ws, cnta, cntb = k1(flat)
    rank, tot = k2(cnta, cntb)

    fref = jax.new_ref(jnp.zeros((FOUT,), F32))
    cref = jax.new_ref(jnp.zeros((COUT,), I32))
    k3(vid, xs, ys, zs, ws, rank, tot, fref, cref)

    features = fref[...].reshape(MAXV, MAXP, 4)
    coors = cref[...].reshape(MAXV, 4)
    return features, coors
